# baseline (speedup 1.0000x reference)
import functools

import jax
import jax.numpy as jnp
from jax import lax
from jax.experimental import pallas as pl
from jax.experimental.pallas import tpu as pltpu
from jax.experimental.pallas import tpu_sc as plsc

D = 64
L = 16
CHUNK = 128


def _recommender_scores(uidx, uoff, iidx, ioff, utab, itab, *,
                        n_workers, b_per_w):
    n_chunks = b_per_w // CHUNK
    mesh = plsc.VectorSubcoreMesh(core_axis_name="c", subcore_axis_name="s")

    @functools.partial(
        pl.kernel,
        mesh=mesh,
        compiler_params=pltpu.CompilerParams(needs_layout_passes=False),
        out_type=jax.ShapeDtypeStruct((n_workers, b_per_w), jnp.float32),
        scratch_types=[
            pltpu.VMEM((n_chunks, CHUNK), jnp.int32),
            pltpu.VMEM((n_chunks, CHUNK), jnp.int32),
            pltpu.VMEM((n_chunks, CHUNK), jnp.int32),
            pltpu.VMEM((n_chunks, CHUNK), jnp.int32),
            pltpu.VMEM((2, CHUNK, 2 * D), jnp.float32),
            pltpu.VMEM((2, CHUNK, 2 * D), jnp.float32),
            pltpu.VMEM((L * b_per_w,), jnp.float32),
            pltpu.VMEM((b_per_w,), jnp.float32),
            pltpu.SemaphoreType.DMA,
            pltpu.SemaphoreType.DMA,
            pltpu.SemaphoreType.DMA,
            pltpu.SemaphoreType.DMA,
        ],
    )
    def k(uidx_hbm, uoff_hbm, iidx_hbm, ioff_hbm, utab_hbm, itab_hbm, out_hbm,
          uidx_v, uoff_v, iidx_v, ioff_v, ubuf, ibuf, tpose_v, out_v,
          usem0, usem1, isem0, isem1):
        wid = lax.axis_index("s") * mesh.num_cores + lax.axis_index("c")
        pltpu.sync_copy(uidx_hbm.at[wid], uidx_v)
        pltpu.sync_copy(uoff_hbm.at[wid], uoff_v)
        pltpu.sync_copy(iidx_hbm.at[wid], iidx_v)
        pltpu.sync_copy(ioff_hbm.at[wid], ioff_v)
        usems = (usem0, usem1)
        isems = (isem0, isem1)

        def fire(j):
            s = j % 2
            pltpu.async_copy(utab_hbm.at[uidx_v.at[j]], ubuf.at[s], usems[s])
            pltpu.async_copy(itab_hbm.at[iidx_v.at[j]], ibuf.at[s], isems[s])

        def drain(j):
            s = j % 2
            pltpu.make_async_copy(utab_hbm.at[uidx_v.at[j]], ubuf.at[s],
                                  usems[s]).wait()
            pltpu.make_async_copy(itab_hbm.at[iidx_v.at[j]], ibuf.at[s],
                                  isems[s]).wait()

        lane_ids = lax.iota(jnp.int32, L)
        fire(0)
        for j in range(n_chunks):
            if j + 1 < n_chunks:
                fire(j + 1)
            drain(j)
            s = j % 2
            tcol_base = lane_ids * b_per_w + j * CHUNK

            @plsc.parallel_loop(0, CHUNK, 1, unroll=8)
            def body(b):
                full_b = jnp.full((L,), b, jnp.int32)
                uo = plsc.load_gather(uoff_v.at[j], [full_b]) + lane_ids
                io = plsc.load_gather(ioff_v.at[j], [full_b]) + lane_ids
                acc = (plsc.load_gather(ubuf.at[s], [full_b, uo]) *
                       plsc.load_gather(ibuf.at[s], [full_b, io]))
                for c in range(1, D // L):
                    acc += (plsc.load_gather(ubuf.at[s], [full_b, uo + c * L]) *
                            plsc.load_gather(ibuf.at[s], [full_b, io + c * L]))
                plsc.store_scatter(tpose_v, [tcol_base + full_b], acc)

        @plsc.parallel_loop(0, b_per_w // L, 1, unroll=2)
        def reduce_body(m):
            acc = tpose_v[pl.ds(m * L, L)]
            for c in range(1, L):
                acc += tpose_v[pl.ds(c * b_per_w + m * L, L)]
            out_v[pl.ds(m * L, L)] = acc

        pltpu.sync_copy(out_v, out_hbm.at[wid])

    return k(uidx, uoff, iidx, ioff, utab, itab)


def kernel(user_ids, item_ids, user_table, item_table):
    b = user_ids.shape[0]
    info = plsc.get_sparse_core_info()
    n_workers = info.num_cores * info.num_subcores
    b_per_w = b // n_workers
    shape3 = (n_workers, b_per_w // CHUNK, CHUNK)
    uids = user_ids.astype(jnp.int32)
    iids = item_ids.astype(jnp.int32)
    uidx = (uids >> 1).reshape(shape3)
    uoff = ((uids & 1) * D).reshape(shape3)
    iidx = (iids >> 1).reshape(shape3)
    ioff = ((iids & 1) * D).reshape(shape3)
    utab = user_table.reshape(user_table.shape[0] // 2, 2 * D)
    itab = item_table.reshape(item_table.shape[0] // 2, 2 * D)
    out = _recommender_scores(uidx, uoff, iidx, ioff, utab, itab,
                              n_workers=n_workers, b_per_w=b_per_w)
    return out.reshape(b)


# speedup vs baseline: 2.3563x; 2.3563x over previous
import functools

import jax
import jax.numpy as jnp
from jax import lax
from jax.experimental import pallas as pl
from jax.experimental.pallas import tpu as pltpu
from jax.experimental.pallas import tpu_sc as plsc

D = 64
L = 16
W = 128


def _recommender_scores(uids, iids, utabT, itabT, *, n_workers, b_per_w):
    mesh = plsc.VectorSubcoreMesh(core_axis_name="c", subcore_axis_name="s")

    @functools.partial(
        pl.kernel,
        mesh=mesh,
        compiler_params=pltpu.CompilerParams(needs_layout_passes=False),
        out_type=jax.ShapeDtypeStruct((n_workers, b_per_w), jnp.float32),
        scratch_types=[
            pltpu.VMEM((b_per_w,), jnp.int32),
            pltpu.VMEM((b_per_w,), jnp.int32),
            pltpu.VMEM((D,), jnp.int32),
            pltpu.VMEM((2, D, W), jnp.float32),
            pltpu.VMEM((2, D, W), jnp.float32),
            pltpu.VMEM((L * b_per_w,), jnp.float32),
            pltpu.VMEM((b_per_w,), jnp.float32),
            pltpu.SemaphoreType.DMA,
            pltpu.SemaphoreType.DMA,
        ],
    )
    def k(uid_hbm, iid_hbm, utab_hbm, itab_hbm, out_hbm,
          uids_v, iids_v, fidx_v, uwin, iwin, tpose_v, out_v, sem0, sem1):
        wid = lax.axis_index("s") * mesh.num_cores + lax.axis_index("c")
        pltpu.sync_copy(uid_hbm.at[wid], uids_v)
        pltpu.sync_copy(iid_hbm.at[wid], iids_v)
        for c in range(D // L):
            fidx_v[pl.ds(c * L, L)] = lax.iota(jnp.int32, L) + c * L
        sems = (sem0, sem1)
        lane_ids = lax.iota(jnp.int32, L)

        def bases(uvec, ivec, k_):
            ub = pl.multiple_of((uvec[k_] >> 7) * W, W)
            ib = pl.multiple_of((ivec[k_] >> 7) * W, W)
            return ub, ib

        def fire(uvec, ivec, k_):
            s = k_ % 2
            ub, ib = bases(uvec, ivec, k_)
            pltpu.async_copy(utab_hbm.at[fidx_v, pl.ds(ub, W)],
                             uwin.at[s], sems[s])
            pltpu.async_copy(itab_hbm.at[fidx_v, pl.ds(ib, W)],
                             iwin.at[s], sems[s])

        def drain(uvec, ivec, k_):
            s = k_ % 2
            ub, ib = bases(uvec, ivec, k_)
            pltpu.make_async_copy(utab_hbm.at[fidx_v, pl.ds(ub, W)],
                                  uwin.at[s], sems[s]).wait()
            pltpu.make_async_copy(itab_hbm.at[fidx_v, pl.ds(ib, W)],
                                  iwin.at[s], sems[s]).wait()

        def compute(uvec, ivec, p, k_):
            s = k_ % 2
            ul = jnp.full((L,), uvec[k_], jnp.int32) & (W - 1)
            il = jnp.full((L,), ivec[k_], jnp.int32) & (W - 1)
            acc = (plsc.load_gather(uwin.at[s], [lane_ids, ul]) *
                   plsc.load_gather(iwin.at[s], [lane_ids, il]))
            for c in range(1, D // L):
                acc += (plsc.load_gather(uwin.at[s], [lane_ids + c * L, ul]) *
                        plsc.load_gather(iwin.at[s], [lane_ids + c * L, il]))
            plsc.store_scatter(tpose_v, [lane_ids * b_per_w + p], acc)

        n_groups = b_per_w // L
        fire(uids_v[pl.ds(0, L)], iids_v[pl.ds(0, L)], 0)

        @pl.loop(0, n_groups)
        def body(g):
            uvec = uids_v[pl.ds(g * L, L)]
            ivec = iids_v[pl.ds(g * L, L)]
            for k_ in range(L):
                if k_ < L - 1:
                    fire(uvec, ivec, k_ + 1)
                else:
                    @pl.when(g + 1 < n_groups)
                    def _():
                        uv2 = uids_v[pl.ds((g + 1) * L, L)]
                        iv2 = iids_v[pl.ds((g + 1) * L, L)]
                        fire(uv2, iv2, 0)

                drain(uvec, ivec, k_)
                compute(uvec, ivec, g * L + k_, k_)

        @plsc.parallel_loop(0, b_per_w // L, 1, unroll=2)
        def reduce_body(m):
            acc = tpose_v[pl.ds(m * L, L)]
            for c in range(1, L):
                acc += tpose_v[pl.ds(c * b_per_w + m * L, L)]
            out_v[pl.ds(m * L, L)] = acc

        pltpu.sync_copy(out_v, out_hbm.at[wid])

    return k(uids, iids, utabT, itabT)


def kernel(user_ids, item_ids, user_table, item_table):
    b = user_ids.shape[0]
    info = plsc.get_sparse_core_info()
    n_workers = info.num_cores * info.num_subcores
    b_per_w = b // n_workers
    uids = user_ids.astype(jnp.int32).reshape(n_workers, b_per_w)
    iids = item_ids.astype(jnp.int32).reshape(n_workers, b_per_w)
    out = _recommender_scores(uids, iids, user_table.T, item_table.T,
                              n_workers=n_workers, b_per_w=b_per_w)
    return out.reshape(b)


# speedup vs baseline: 2.6670x; 1.1319x over previous
import functools

import jax
import jax.numpy as jnp
from jax import lax
from jax.experimental import pallas as pl
from jax.experimental.pallas import tpu as pltpu
from jax.experimental.pallas import tpu_sc as plsc

D = 64
L = 16
W = 128
NBUF = 4


def _recommender_scores(uids, iids, utabT, itabT, *, n_workers, b_per_w):
    mesh = plsc.VectorSubcoreMesh(core_axis_name="c", subcore_axis_name="s")

    @functools.partial(
        pl.kernel,
        mesh=mesh,
        compiler_params=pltpu.CompilerParams(needs_layout_passes=False),
        out_type=jax.ShapeDtypeStruct((n_workers, b_per_w), jnp.float32),
        scratch_types=[
            pltpu.VMEM((b_per_w,), jnp.int32),
            pltpu.VMEM((b_per_w,), jnp.int32),
            pltpu.VMEM((D,), jnp.int32),
            pltpu.VMEM((NBUF, D, W), jnp.float32),
            pltpu.VMEM((NBUF, D, W), jnp.float32),
            pltpu.VMEM((L * b_per_w,), jnp.float32),
            pltpu.VMEM((b_per_w,), jnp.float32),
            pltpu.SemaphoreType.DMA,
            pltpu.SemaphoreType.DMA,
            pltpu.SemaphoreType.DMA,
            pltpu.SemaphoreType.DMA,
        ],
    )
    def k(uid_hbm, iid_hbm, utab_hbm, itab_hbm, out_hbm,
          uids_v, iids_v, fidx_v, uwin, iwin, tpose_v, out_v,
          sem0, sem1, sem2, sem3):
        wid = lax.axis_index("s") * mesh.num_cores + lax.axis_index("c")
        pltpu.sync_copy(uid_hbm.at[wid], uids_v)
        pltpu.sync_copy(iid_hbm.at[wid], iids_v)
        for c in range(D // L):
            fidx_v[pl.ds(c * L, L)] = lax.iota(jnp.int32, L) + c * L
        sems = (sem0, sem1, sem2, sem3)
        lane_ids = lax.iota(jnp.int32, L)

        def bases(uvec, ivec, k_):
            ub = pl.multiple_of((uvec[k_] >> 7) * W, W)
            ib = pl.multiple_of((ivec[k_] >> 7) * W, W)
            return ub, ib

        def fire(uvec, ivec, k_):
            s = k_ % NBUF
            ub, ib = bases(uvec, ivec, k_)
            pltpu.async_copy(utab_hbm.at[fidx_v, pl.ds(ub, W)],
                             uwin.at[s], sems[s])
            pltpu.async_copy(itab_hbm.at[fidx_v, pl.ds(ib, W)],
                             iwin.at[s], sems[s])

        def drain(uvec, ivec, k_):
            s = k_ % NBUF
            ub, ib = bases(uvec, ivec, k_)
            pltpu.make_async_copy(utab_hbm.at[fidx_v, pl.ds(ub, W)],
                                  uwin.at[s], sems[s]).wait()
            pltpu.make_async_copy(itab_hbm.at[fidx_v, pl.ds(ib, W)],
                                  iwin.at[s], sems[s]).wait()

        def compute(uvec, ivec, p, k_):
            s = k_ % NBUF
            ul = jnp.full((L,), uvec[k_], jnp.int32) & (W - 1)
            il = jnp.full((L,), ivec[k_], jnp.int32) & (W - 1)
            acc = (plsc.load_gather(uwin.at[s], [lane_ids, ul]) *
                   plsc.load_gather(iwin.at[s], [lane_ids, il]))
            for c in range(1, D // L):
                acc += (plsc.load_gather(uwin.at[s], [lane_ids + c * L, ul]) *
                        plsc.load_gather(iwin.at[s], [lane_ids + c * L, il]))
            plsc.store_scatter(tpose_v, [lane_ids * b_per_w + p], acc)

        n_groups = b_per_w // L
        uvec0 = uids_v[pl.ds(0, L)]
        ivec0 = iids_v[pl.ds(0, L)]
        for k_ in range(NBUF - 1):
            fire(uvec0, ivec0, k_)

        @pl.loop(0, n_groups)
        def body(g):
            uvec = uids_v[pl.ds(g * L, L)]
            ivec = iids_v[pl.ds(g * L, L)]
            for k_ in range(L):
                ahead = k_ + NBUF - 1
                if ahead < L:
                    fire(uvec, ivec, ahead)
                else:
                    @pl.when(g + 1 < n_groups)
                    def _():
                        uv2 = uids_v[pl.ds((g + 1) * L, L)]
                        iv2 = iids_v[pl.ds((g + 1) * L, L)]
                        fire(uv2, iv2, ahead - L)

                drain(uvec, ivec, k_)
                compute(uvec, ivec, g * L + k_, k_)

        @plsc.parallel_loop(0, b_per_w // L, 1, unroll=2)
        def reduce_body(m):
            acc = tpose_v[pl.ds(m * L, L)]
            for c in range(1, L):
                acc += tpose_v[pl.ds(c * b_per_w + m * L, L)]
            out_v[pl.ds(m * L, L)] = acc

        pltpu.sync_copy(out_v, out_hbm.at[wid])

    return k(uids, iids, utabT, itabT)


def kernel(user_ids, item_ids, user_table, item_table):
    b = user_ids.shape[0]
    info = plsc.get_sparse_core_info()
    n_workers = info.num_cores * info.num_subcores
    b_per_w = b // n_workers
    uids = user_ids.astype(jnp.int32).reshape(n_workers, b_per_w)
    iids = item_ids.astype(jnp.int32).reshape(n_workers, b_per_w)
    out = _recommender_scores(uids, iids, user_table.T, item_table.T,
                              n_workers=n_workers, b_per_w=b_per_w)
    return out.reshape(b)
